# baseline (speedup 1.0000x reference)
import functools

import jax
import jax.numpy as jnp
from jax import lax
from jax.experimental import pallas as pl
from jax.experimental.pallas import tpu as pltpu
from jax.experimental.pallas import tpu_sc as plsc

N = 10000
E = 160000
D = 256
DH = 128
NC = 2
NS = 16
CHUNK = 128
CPT = 80
ET = NS * CPT * CHUNK
HCPT = 40
TILE_ROWS = 624
TAIL_ROWS = N - (NS - 1) * TILE_ROWS
EPS = 1e-5


def _fill2d(ref, nrows, ncols, val):
    vec = jnp.full((16,), val, jnp.float32)

    def row(i, _):
        def col(j, _):
            ref[i, pl.ds(j * 16, 16)] = vec
            return 0

        lax.fori_loop(0, ncols // 16, col, 0)
        return 0

    lax.fori_loop(0, nrows, row, 0)


def _zero_stripe(src_buf, sh, base, s):
    for off in (0, 128, 256, 384):
        pltpu.sync_copy(src_buf, sh.at[pl.ds(base + off, 128)])
    pltpu.sync_copy(src_buf.at[pl.ds(0, 112)], sh.at[pl.ds(base + 512, 112)])

    @pl.when(s == NS - 1)
    def _():
        pltpu.sync_copy(src_buf.at[pl.ds(0, 24)],
                        sh.at[pl.ds(NS * TILE_ROWS, 24)])


def _write_stripe(sh, out, base, out_row0, s):
    out_base = pl.multiple_of(out_row0 + base, 8)

    @pl.when(s < NS - 1)
    def _():
        pltpu.sync_copy(sh.at[pl.ds(base, TILE_ROWS)],
                        out.at[pl.ds(out_base, TILE_ROWS)])

    @pl.when(s == NS - 1)
    def _():
        pltpu.sync_copy(sh.at[pl.ds(base, TAIL_ROWS)],
                        out.at[pl.ds(out_base, TAIL_ROWS)])


def _cnt_body(dst2, cnt_out, cnt_sh, didx2, ones, ssem):
    c = lax.axis_index("c")
    s = lax.axis_index("s")
    base = pl.multiple_of(s * TILE_ROWS, 8)

    pltpu.sync_copy(dst2.at[pl.ds(s * CPT, CPT)], didx2)
    _fill2d(ones, CHUNK, DH, 0.0)
    _zero_stripe(ones, cnt_sh, base, s)
    _fill2d(ones, CHUNK, DH, 1.0)
    plsc.subcore_barrier()

    def group(g, _):
        descs = [
            pltpu.async_copy(ones, cnt_sh.at[didx2.at[2 * (8 * g + j) + c]],
                             ssem, add=True)
            for j in range(8)
        ]
        for dsc in descs:
            dsc.wait()
        return 0

    lax.fori_loop(0, CPT // 16, group, 0)
    plsc.subcore_barrier()
    _write_stripe(cnt_sh, cnt_out, base, c * N, s)


_sc_cnt = functools.partial(
    pl.kernel,
    out_type=jax.ShapeDtypeStruct((NC * N, DH), jnp.float32),
    mesh=plsc.VectorSubcoreMesh(core_axis_name="c", subcore_axis_name="s",
                                num_cores=NC, num_subcores=NS),
    scratch_types=[
        pltpu.VMEM_SHARED((N + 8, DH), jnp.float32),
        pltpu.VMEM((CPT, CHUNK), jnp.int32),
        pltpu.VMEM((CHUNK, DH), jnp.float32),
        pltpu.SemaphoreType.DMA,
    ],
)(_cnt_body)


def _agg_body(x0, x1, src2, dst2, agg_out, agg_sh, sidx2, didx2, rows0, rows1,
              sem0, sem1, ssem0, ssem1):
    c = lax.axis_index("c")
    s = lax.axis_index("s")
    base = pl.multiple_of(s * TILE_ROWS, 8)

    _fill2d(rows0, CHUNK, DH, 0.0)
    _zero_stripe(rows0, agg_sh, base, s)
    plsc.subcore_barrier()

    def work(xh):
        def g_wait(buf, sem):
            pltpu.make_async_copy(xh.at[sidx2.at[0]], buf, sem).wait()

        def s_wait(buf, sem):
            pltpu.make_async_copy(buf, agg_sh.at[didx2.at[0]], sem).wait()

        for h in range(CPT // HCPT):
            row0 = pl.multiple_of(s * CPT + h * HCPT, 8)
            pltpu.sync_copy(src2.at[pl.ds(row0, HCPT)], sidx2)
            pltpu.sync_copy(dst2.at[pl.ds(row0, HCPT)], didx2)
            pltpu.async_copy(xh.at[sidx2.at[0]], rows0, sem0)
            pltpu.async_copy(xh.at[sidx2.at[1]], rows1, sem1)

            def step(k2, _):
                k = 2 * k2
                g_wait(rows0, sem0)
                pltpu.async_copy(rows0, agg_sh.at[didx2.at[k]], ssem0,
                                 add=True)
                g_wait(rows1, sem1)
                pltpu.async_copy(rows1, agg_sh.at[didx2.at[k + 1]], ssem1,
                                 add=True)
                s_wait(rows0, ssem0)
                s_wait(rows1, ssem1)

                @pl.when(k + 2 < HCPT)
                def _():
                    pltpu.async_copy(xh.at[sidx2.at[k + 2]], rows0, sem0)
                    pltpu.async_copy(xh.at[sidx2.at[k + 3]], rows1, sem1)

                return 0

            lax.fori_loop(0, HCPT // 2, step, 0)

    @pl.when(c == 0)
    def _():
        work(x0)

    @pl.when(c == 1)
    def _():
        work(x1)

    plsc.subcore_barrier()
    _write_stripe(agg_sh, agg_out, base, c * N, s)


_sc_agg = functools.partial(
    pl.kernel,
    out_type=jax.ShapeDtypeStruct((NC * N, DH), jnp.float32),
    mesh=plsc.VectorSubcoreMesh(core_axis_name="c", subcore_axis_name="s",
                                num_cores=NC, num_subcores=NS),
    scratch_types=[
        pltpu.VMEM_SHARED((N + 8, DH), jnp.float32),
        pltpu.VMEM((HCPT, CHUNK), jnp.int32),
        pltpu.VMEM((HCPT, CHUNK), jnp.int32),
        pltpu.VMEM((CHUNK, DH), jnp.float32),
        pltpu.VMEM((CHUNK, DH), jnp.float32),
        pltpu.SemaphoreType.DMA,
        pltpu.SemaphoreType.DMA,
        pltpu.SemaphoreType.DMA,
        pltpu.SemaphoreType.DMA,
    ],
)(_agg_body)


NB = 5
BR = N // NB


def _tc_body(x_ref, a0_ref, a1_ref, cnt0_ref, cnt1_ref, wl_ref, bl_ref,
             wr_ref, g_ref, b_ref, o_ref, h_scr, sum_scr, sq_scr):
    p = pl.program_id(0)
    i = pl.program_id(1)

    @pl.when(p == 0)
    def _():
        cnt = cnt0_ref[:, 0:1] + cnt1_ref[:, 0:1]
        inv = 1.0 / jnp.maximum(cnt, 1.0)
        t = jnp.dot(a0_ref[...], wl_ref[0:DH, :],
                    preferred_element_type=jnp.float32,
                    precision=lax.Precision.DEFAULT)
        t = t + jnp.dot(a1_ref[...], wl_ref[DH:D, :],
                        preferred_element_type=jnp.float32,
                        precision=lax.Precision.DEFAULT)
        h = t * inv + jnp.dot(x_ref[...], wr_ref[...],
                              preferred_element_type=jnp.float32,
                              precision=lax.Precision.DEFAULT) + bl_ref[...]
        h_scr[pl.ds(i * BR, BR), :] = h
        csum = jnp.sum(h, axis=0, keepdims=True)
        csq = jnp.sum(h * h, axis=0, keepdims=True)

        @pl.when(i == 0)
        def _():
            sum_scr[...] = csum
            sq_scr[...] = csq

        @pl.when(i > 0)
        def _():
            sum_scr[...] = sum_scr[...] + csum
            sq_scr[...] = sq_scr[...] + csq

    @pl.when(p == 1)
    def _():
        mu = sum_scr[...] * (1.0 / N)
        var = sq_scr[...] * (1.0 / N) - mu * mu
        scale = lax.rsqrt(var + EPS) * g_ref[...]
        h = h_scr[pl.ds(i * BR, BR), :]
        o_ref[...] = jnp.maximum((h - mu) * scale + b_ref[...], 0.0)


def kernel(x, edge_index, W_l, b_l, W_r, gamma, beta):
    pad = ET - E
    src = jnp.concatenate([edge_index[0], jnp.zeros((pad,), jnp.int32)])
    dst = jnp.concatenate(
        [edge_index[1], N + (jnp.arange(pad, dtype=jnp.int32) % 8)])
    src2 = src.reshape(ET // CHUNK, CHUNK)
    dst2 = dst.reshape(ET // CHUNK, CHUNK)
    x0 = x[:, :DH]
    x1 = x[:, DH:]
    cnt = _sc_cnt(dst2)
    agg = _sc_agg(x0, x1, src2, dst2)
    row_blk = lambda p, i: (i, 0)
    return pl.pallas_call(
        _tc_body,
        grid=(2, NB),
        in_specs=[
            pl.BlockSpec((BR, D), row_blk),
            pl.BlockSpec((BR, DH), row_blk),
            pl.BlockSpec((BR, DH), lambda p, i: (NB + i, 0)),
            pl.BlockSpec((BR, DH), row_blk),
            pl.BlockSpec((BR, DH), lambda p, i: (NB + i, 0)),
            pl.BlockSpec((D, D), lambda p, i: (0, 0)),
            pl.BlockSpec((1, D), lambda p, i: (0, 0)),
            pl.BlockSpec((D, D), lambda p, i: (0, 0)),
            pl.BlockSpec((1, D), lambda p, i: (0, 0)),
            pl.BlockSpec((1, D), lambda p, i: (0, 0)),
        ],
        out_specs=pl.BlockSpec((BR, D), row_blk),
        out_shape=jax.ShapeDtypeStruct((N, D), jnp.float32),
        scratch_shapes=[
            pltpu.VMEM((N, D), jnp.float32),
            pltpu.VMEM((1, D), jnp.float32),
            pltpu.VMEM((1, D), jnp.float32),
        ],
    )(x, agg, agg, cnt, cnt, W_l, b_l.reshape(1, D), W_r,
      gamma.reshape(1, D), beta.reshape(1, D))


# speedup vs baseline: 1.0275x; 1.0275x over previous
import functools

import jax
import jax.numpy as jnp
from jax import lax
from jax.experimental import pallas as pl
from jax.experimental.pallas import tpu as pltpu
from jax.experimental.pallas import tpu_sc as plsc

N = 10000
E = 160000
D = 256
DH = 128
NC = 2
NS = 16
CHUNK = 80
CPT = 128
ET = NS * CPT * CHUNK
SEG = 16
NBUF = 4
TILE_ROWS = 624
TAIL_ROWS = N - (NS - 1) * TILE_ROWS
EPS = 1e-5


def _fill2d(ref, nrows, ncols, val):
    vec = jnp.full((16,), val, jnp.float32)

    def row(i, _):
        def col(j, _):
            ref[i, pl.ds(j * 16, 16)] = vec
            return 0

        lax.fori_loop(0, ncols // 16, col, 0)
        return 0

    lax.fori_loop(0, nrows, row, 0)


def _zero_stripe(src_buf, nrows_buf, sh, base, s):
    full = TILE_ROWS // nrows_buf
    rem = TILE_ROWS - full * nrows_buf
    for f in range(full):
        pltpu.sync_copy(src_buf, sh.at[pl.ds(base + f * nrows_buf, nrows_buf)])
    if rem:
        pltpu.sync_copy(src_buf.at[pl.ds(0, rem)],
                        sh.at[pl.ds(base + full * nrows_buf, rem)])

    @pl.when(s == NS - 1)
    def _():
        pltpu.sync_copy(src_buf.at[pl.ds(0, 24)],
                        sh.at[pl.ds(NS * TILE_ROWS, 24)])


def _write_stripe(sh, out, base, out_row0, s):
    out_base = pl.multiple_of(out_row0 + base, 8)

    @pl.when(s < NS - 1)
    def _():
        pltpu.sync_copy(sh.at[pl.ds(base, TILE_ROWS)],
                        out.at[pl.ds(out_base, TILE_ROWS)])

    @pl.when(s == NS - 1)
    def _():
        pltpu.sync_copy(sh.at[pl.ds(base, TAIL_ROWS)],
                        out.at[pl.ds(out_base, TAIL_ROWS)])


def _cnt_body(dst2, cnt_out, cnt_sh, didx2, ones, ssem):
    c = lax.axis_index("c")
    s = lax.axis_index("s")
    base = pl.multiple_of(s * TILE_ROWS, 8)

    pltpu.sync_copy(dst2.at[pl.ds(s * CPT, CPT)], didx2)
    _fill2d(ones, CHUNK, DH, 0.0)
    _zero_stripe(ones, CHUNK, cnt_sh, base, s)
    _fill2d(ones, CHUNK, DH, 1.0)
    plsc.subcore_barrier()

    def group(g, _):
        descs = [
            pltpu.async_copy(ones, cnt_sh.at[didx2.at[2 * (8 * g + j) + c]],
                             ssem, add=True)
            for j in range(8)
        ]
        for dsc in descs:
            dsc.wait()
        return 0

    lax.fori_loop(0, CPT // 16, group, 0)
    plsc.subcore_barrier()
    _write_stripe(cnt_sh, cnt_out, base, c * N, s)


_sc_cnt = functools.partial(
    pl.kernel,
    out_type=jax.ShapeDtypeStruct((NC * N, DH), jnp.float32),
    mesh=plsc.VectorSubcoreMesh(core_axis_name="c", subcore_axis_name="s",
                                num_cores=NC, num_subcores=NS),
    scratch_types=[
        pltpu.VMEM_SHARED((N + 8, DH), jnp.float32),
        pltpu.VMEM((CPT, CHUNK), jnp.int32),
        pltpu.VMEM((CHUNK, DH), jnp.float32),
        pltpu.SemaphoreType.DMA,
    ],
)(_cnt_body)


def _agg_body(x0, x1, src2, dst2, agg_out, agg_sh, sidx2, didx2,
              rows0, rows1, rows2, rows3,
              gsem0, gsem1, gsem2, gsem3, ssem0, ssem1, ssem2, ssem3):
    c = lax.axis_index("c")
    s = lax.axis_index("s")
    base = pl.multiple_of(s * TILE_ROWS, 8)
    rows = (rows0, rows1, rows2, rows3)
    gsems = (gsem0, gsem1, gsem2, gsem3)
    ssems = (ssem0, ssem1, ssem2, ssem3)

    _fill2d(rows0, CHUNK, DH, 0.0)
    _zero_stripe(rows0, CHUNK, agg_sh, base, s)
    plsc.subcore_barrier()

    def work(xh):
        def g_wait(b):
            pltpu.make_async_copy(xh.at[sidx2.at[0]], rows[b], gsems[b]).wait()

        def s_wait(b):
            pltpu.make_async_copy(rows[b], agg_sh.at[didx2.at[0]],
                                  ssems[b]).wait()

        for seg in range(CPT // SEG):
            row0 = pl.multiple_of(s * CPT + seg * SEG, 8)
            pltpu.sync_copy(src2.at[pl.ds(row0, SEG)], sidx2)
            pltpu.sync_copy(dst2.at[pl.ds(row0, SEG)], didx2)
            for b in range(NBUF):
                pltpu.async_copy(xh.at[sidx2.at[b]], rows[b], gsems[b])

            def rnd(j, _):
                for b in range(NBUF):
                    g_wait(b)
                    pltpu.async_copy(rows[b],
                                     agg_sh.at[didx2.at[NBUF * j + b]],
                                     ssems[b], add=True)
                for b in range(NBUF):
                    s_wait(b)

                    @pl.when(NBUF * (j + 1) + b < SEG)
                    def _():
                        pltpu.async_copy(xh.at[sidx2.at[NBUF * (j + 1) + b]],
                                         rows[b], gsems[b])

                return 0

            lax.fori_loop(0, SEG // NBUF, rnd, 0)

    @pl.when(c == 0)
    def _():
        work(x0)

    @pl.when(c == 1)
    def _():
        work(x1)

    plsc.subcore_barrier()
    _write_stripe(agg_sh, agg_out, base, c * N, s)


_sc_agg = functools.partial(
    pl.kernel,
    out_type=jax.ShapeDtypeStruct((NC * N, DH), jnp.float32),
    mesh=plsc.VectorSubcoreMesh(core_axis_name="c", subcore_axis_name="s",
                                num_cores=NC, num_subcores=NS),
    scratch_types=[
        pltpu.VMEM_SHARED((N + 8, DH), jnp.float32),
        pltpu.VMEM((SEG, CHUNK), jnp.int32),
        pltpu.VMEM((SEG, CHUNK), jnp.int32),
        pltpu.VMEM((CHUNK, DH), jnp.float32),
        pltpu.VMEM((CHUNK, DH), jnp.float32),
        pltpu.VMEM((CHUNK, DH), jnp.float32),
        pltpu.VMEM((CHUNK, DH), jnp.float32),
        pltpu.SemaphoreType.DMA,
        pltpu.SemaphoreType.DMA,
        pltpu.SemaphoreType.DMA,
        pltpu.SemaphoreType.DMA,
        pltpu.SemaphoreType.DMA,
        pltpu.SemaphoreType.DMA,
        pltpu.SemaphoreType.DMA,
        pltpu.SemaphoreType.DMA,
    ],
)(_agg_body)


NB = 5
BR = N // NB


def _tc_body(x_ref, a0_ref, a1_ref, cnt0_ref, cnt1_ref, wl_ref, bl_ref,
             wr_ref, g_ref, b_ref, o_ref, h_scr, sum_scr, sq_scr):
    p = pl.program_id(0)
    i = pl.program_id(1)

    @pl.when(p == 0)
    def _():
        cnt = cnt0_ref[:, 0:1] + cnt1_ref[:, 0:1]
        inv = 1.0 / jnp.maximum(cnt, 1.0)
        t = jnp.dot(a0_ref[...], wl_ref[0:DH, :],
                    preferred_element_type=jnp.float32)
        t = t + jnp.dot(a1_ref[...], wl_ref[DH:D, :],
                        preferred_element_type=jnp.float32)
        h = t * inv + jnp.dot(x_ref[...], wr_ref[...],
                              preferred_element_type=jnp.float32) + bl_ref[...]
        h_scr[pl.ds(i * BR, BR), :] = h
        csum = jnp.sum(h, axis=0, keepdims=True)
        csq = jnp.sum(h * h, axis=0, keepdims=True)

        @pl.when(i == 0)
        def _():
            sum_scr[...] = csum
            sq_scr[...] = csq

        @pl.when(i > 0)
        def _():
            sum_scr[...] = sum_scr[...] + csum
            sq_scr[...] = sq_scr[...] + csq

    @pl.when(p == 1)
    def _():
        mu = sum_scr[...] * (1.0 / N)
        var = sq_scr[...] * (1.0 / N) - mu * mu
        scale = lax.rsqrt(var + EPS) * g_ref[...]
        h = h_scr[pl.ds(i * BR, BR), :]
        o_ref[...] = jnp.maximum((h - mu) * scale + b_ref[...], 0.0)


def kernel(x, edge_index, W_l, b_l, W_r, gamma, beta):
    pad = ET - E
    src = jnp.concatenate([edge_index[0], jnp.zeros((pad,), jnp.int32)])
    dst = jnp.concatenate(
        [edge_index[1], N + (jnp.arange(pad, dtype=jnp.int32) % 8)])
    src2 = src.reshape(ET // CHUNK, CHUNK)
    dst2 = dst.reshape(ET // CHUNK, CHUNK)
    x0 = x[:, :DH]
    x1 = x[:, DH:]
    cnt = _sc_cnt(dst2)
    agg = _sc_agg(x0, x1, src2, dst2)
    row_blk = lambda p, i: (i, 0)
    return pl.pallas_call(
        _tc_body,
        grid=(2, NB),
        in_specs=[
            pl.BlockSpec((BR, D), row_blk),
            pl.BlockSpec((BR, DH), row_blk),
            pl.BlockSpec((BR, DH), lambda p, i: (NB + i, 0)),
            pl.BlockSpec((BR, DH), row_blk),
            pl.BlockSpec((BR, DH), lambda p, i: (NB + i, 0)),
            pl.BlockSpec((D, D), lambda p, i: (0, 0)),
            pl.BlockSpec((1, D), lambda p, i: (0, 0)),
            pl.BlockSpec((D, D), lambda p, i: (0, 0)),
            pl.BlockSpec((1, D), lambda p, i: (0, 0)),
            pl.BlockSpec((1, D), lambda p, i: (0, 0)),
        ],
        out_specs=pl.BlockSpec((BR, D), row_blk),
        out_shape=jax.ShapeDtypeStruct((N, D), jnp.float32),
        scratch_shapes=[
            pltpu.VMEM((N, D), jnp.float32),
            pltpu.VMEM((1, D), jnp.float32),
            pltpu.VMEM((1, D), jnp.float32),
        ],
    )(x, agg, agg, cnt, cnt, W_l, b_l.reshape(1, D), W_r,
      gamma.reshape(1, D), beta.reshape(1, D))


# speedup vs baseline: 1.0714x; 1.0427x over previous
import functools

import jax
import jax.numpy as jnp
from jax import lax
from jax.experimental import pallas as pl
from jax.experimental.pallas import tpu as pltpu
from jax.experimental.pallas import tpu_sc as plsc

N = 10000
E = 160000
D = 256
DH = 128
NC = 2
NS = 16
CHUNK = 128
CPT = 80
ET = NS * CPT * CHUNK
SEG = 16
TILE_ROWS = 624
TAIL_ROWS = N - (NS - 1) * TILE_ROWS
EPS = 1e-5


def _fill2d(ref, nrows, ncols, val):
    vec = jnp.full((16,), val, jnp.float32)

    def row(i, _):
        def col(j, _):
            ref[i, pl.ds(j * 16, 16)] = vec
            return 0

        lax.fori_loop(0, ncols // 16, col, 0)
        return 0

    lax.fori_loop(0, nrows, row, 0)


def _zero_stripe(src_buf, sh, base, s):
    for off in (0, 128, 256, 384):
        pltpu.sync_copy(src_buf, sh.at[pl.ds(base + off, 128)])
    pltpu.sync_copy(src_buf.at[pl.ds(0, 112)], sh.at[pl.ds(base + 512, 112)])

    @pl.when(s == NS - 1)
    def _():
        pltpu.sync_copy(src_buf.at[pl.ds(0, 24)],
                        sh.at[pl.ds(NS * TILE_ROWS, 24)])


def _write_stripe(sh, out, base, out_row0, s):
    out_base = pl.multiple_of(out_row0 + base, 8)

    @pl.when(s < NS - 1)
    def _():
        pltpu.sync_copy(sh.at[pl.ds(base, TILE_ROWS)],
                        out.at[pl.ds(out_base, TILE_ROWS)])

    @pl.when(s == NS - 1)
    def _():
        pltpu.sync_copy(sh.at[pl.ds(base, TAIL_ROWS)],
                        out.at[pl.ds(out_base, TAIL_ROWS)])


def _cnt_body(dst2, cnt_out, cnt_sh, didx2, ones, ssem):
    c = lax.axis_index("c")
    s = lax.axis_index("s")
    base = pl.multiple_of(s * TILE_ROWS, 8)

    pltpu.sync_copy(dst2.at[pl.ds(s * CPT, CPT)], didx2)
    _fill2d(ones, CHUNK, DH, 0.0)
    _zero_stripe(ones, cnt_sh, base, s)
    _fill2d(ones, CHUNK, DH, 1.0)
    plsc.subcore_barrier()

    def group(g, _):
        descs = [
            pltpu.async_copy(ones, cnt_sh.at[didx2.at[2 * (8 * g + j) + c]],
                             ssem, add=True)
            for j in range(8)
        ]
        for dsc in descs:
            dsc.wait()
        return 0

    lax.fori_loop(0, CPT // 16, group, 0)
    plsc.subcore_barrier()
    _write_stripe(cnt_sh, cnt_out, base, c * N, s)


_sc_cnt = functools.partial(
    pl.kernel,
    out_type=jax.ShapeDtypeStruct((NC * N, DH), jnp.float32),
    mesh=plsc.VectorSubcoreMesh(core_axis_name="c", subcore_axis_name="s",
                                num_cores=NC, num_subcores=NS),
    scratch_types=[
        pltpu.VMEM_SHARED((N + 8, DH), jnp.float32),
        pltpu.VMEM((CPT, CHUNK), jnp.int32),
        pltpu.VMEM((CHUNK, DH), jnp.float32),
        pltpu.SemaphoreType.DMA,
    ],
)(_cnt_body)


def _agg_body(x0, x1, src2, dst2, agg_out, agg_sh, sidx2, didx2,
              rows0, rows1, sem0, sem1):
    c = lax.axis_index("c")
    s = lax.axis_index("s")
    base = pl.multiple_of(s * TILE_ROWS, 8)

    _fill2d(rows0, CHUNK, DH, 0.0)
    _zero_stripe(rows0, agg_sh, base, s)
    plsc.subcore_barrier()

    def work(xh):
        for h in range(CPT // SEG):
            row0 = pl.multiple_of(s * CPT + h * SEG, 8)
            pltpu.sync_copy(src2.at[pl.ds(row0, SEG)], sidx2)
            pltpu.sync_copy(dst2.at[pl.ds(row0, SEG)], didx2)
            pltpu.async_copy(xh.at[sidx2.at[0]], rows0, sem0)

            def step(k2, _):
                k = 2 * k2
                pltpu.async_copy(xh.at[sidx2.at[k + 1]], rows1, sem1)
                pltpu.make_async_copy(xh.at[sidx2.at[0]], rows0, sem0).wait()
                pltpu.sync_copy(rows0, agg_sh.at[didx2.at[k]], add=True)

                @pl.when(k + 2 < SEG)
                def _():
                    pltpu.async_copy(xh.at[sidx2.at[k + 2]], rows0, sem0)

                pltpu.make_async_copy(xh.at[sidx2.at[0]], rows1, sem1).wait()
                pltpu.sync_copy(rows1, agg_sh.at[didx2.at[k + 1]], add=True)
                return 0

            lax.fori_loop(0, SEG // 2, step, 0)

    @pl.when(c == 0)
    def _():
        work(x0)

    @pl.when(c == 1)
    def _():
        work(x1)

    plsc.subcore_barrier()
    _write_stripe(agg_sh, agg_out, base, c * N, s)


_sc_agg = functools.partial(
    pl.kernel,
    out_type=jax.ShapeDtypeStruct((NC * N, DH), jnp.float32),
    mesh=plsc.VectorSubcoreMesh(core_axis_name="c", subcore_axis_name="s",
                                num_cores=NC, num_subcores=NS),
    scratch_types=[
        pltpu.VMEM_SHARED((N + 8, DH), jnp.float32),
        pltpu.VMEM((SEG, CHUNK), jnp.int32),
        pltpu.VMEM((SEG, CHUNK), jnp.int32),
        pltpu.VMEM((CHUNK, DH), jnp.float32),
        pltpu.VMEM((CHUNK, DH), jnp.float32),
        pltpu.SemaphoreType.DMA,
        pltpu.SemaphoreType.DMA,
    ],
)(_agg_body)


NB = 5
BR = N // NB


def _tcr_body(x_ref, wr_ref, bl_ref, r_ref):
    r_ref[...] = jnp.dot(x_ref[...], wr_ref[...],
                         preferred_element_type=jnp.float32) + bl_ref[...]


def _tc_body(r_ref, a0_ref, a1_ref, cnt0_ref, cnt1_ref, wl_ref, g_ref, b_ref,
             o_ref, h_scr, sum_scr, sq_scr):
    p = pl.program_id(0)
    i = pl.program_id(1)

    @pl.when(p == 0)
    def _():
        cnt = cnt0_ref[:, 0:1] + cnt1_ref[:, 0:1]
        inv = 1.0 / jnp.maximum(cnt, 1.0)
        t = jnp.dot(a0_ref[...], wl_ref[0:DH, :],
                    preferred_element_type=jnp.float32)
        t = t + jnp.dot(a1_ref[...], wl_ref[DH:D, :],
                        preferred_element_type=jnp.float32)
        h = t * inv + r_ref[...]
        h_scr[pl.ds(i * BR, BR), :] = h
        csum = jnp.sum(h, axis=0, keepdims=True)
        csq = jnp.sum(h * h, axis=0, keepdims=True)

        @pl.when(i == 0)
        def _():
            sum_scr[...] = csum
            sq_scr[...] = csq

        @pl.when(i > 0)
        def _():
            sum_scr[...] = sum_scr[...] + csum
            sq_scr[...] = sq_scr[...] + csq

    @pl.when(p == 1)
    def _():
        mu = sum_scr[...] * (1.0 / N)
        var = sq_scr[...] * (1.0 / N) - mu * mu
        scale = lax.rsqrt(var + EPS) * g_ref[...]
        h = h_scr[pl.ds(i * BR, BR), :]
        o_ref[...] = jnp.maximum((h - mu) * scale + b_ref[...], 0.0)


def kernel(x, edge_index, W_l, b_l, W_r, gamma, beta):
    pad = ET - E
    src = jnp.concatenate([edge_index[0], jnp.zeros((pad,), jnp.int32)])
    dst = jnp.concatenate(
        [edge_index[1], N + (jnp.arange(pad, dtype=jnp.int32) % 8)])
    src2 = src.reshape(ET // CHUNK, CHUNK)
    dst2 = dst.reshape(ET // CHUNK, CHUNK)
    x0 = x[:, :DH]
    x1 = x[:, DH:]
    cnt = _sc_cnt(dst2)
    agg = _sc_agg(x0, x1, src2, dst2)
    row_blk = lambda p, i: (i, 0)
    r = pl.pallas_call(
        _tcr_body,
        grid=(NB,),
        in_specs=[
            pl.BlockSpec((BR, D), lambda i: (i, 0)),
            pl.BlockSpec((D, D), lambda i: (0, 0)),
            pl.BlockSpec((1, D), lambda i: (0, 0)),
        ],
        out_specs=pl.BlockSpec((BR, D), lambda i: (i, 0)),
        out_shape=jax.ShapeDtypeStruct((N, D), jnp.float32),
    )(x, W_r, b_l.reshape(1, D))
    return pl.pallas_call(
        _tc_body,
        grid=(2, NB),
        in_specs=[
            pl.BlockSpec((BR, D), row_blk),
            pl.BlockSpec((BR, DH), row_blk),
            pl.BlockSpec((BR, DH), lambda p, i: (NB + i, 0)),
            pl.BlockSpec((BR, DH), row_blk),
            pl.BlockSpec((BR, DH), lambda p, i: (NB + i, 0)),
            pl.BlockSpec((D, D), lambda p, i: (0, 0)),
            pl.BlockSpec((1, D), lambda p, i: (0, 0)),
            pl.BlockSpec((1, D), lambda p, i: (0, 0)),
        ],
        out_specs=pl.BlockSpec((BR, D), row_blk),
        out_shape=jax.ShapeDtypeStruct((N, D), jnp.float32),
        scratch_shapes=[
            pltpu.VMEM((N, D), jnp.float32),
            pltpu.VMEM((1, D), jnp.float32),
            pltpu.VMEM((1, D), jnp.float32),
        ],
    )(r, agg, agg, cnt, cnt, W_l, gamma.reshape(1, D), beta.reshape(1, D))


# speedup vs baseline: 1.0995x; 1.0262x over previous
import functools

import jax
import jax.numpy as jnp
from jax import lax
from jax.experimental import pallas as pl
from jax.experimental.pallas import tpu as pltpu
from jax.experimental.pallas import tpu_sc as plsc

N = 10000
E = 160000
D = 256
DH = 128
NC = 2
NS = 16
CHUNK = 128
CPT = 80
ET = NS * CPT * CHUNK
SEG = 40
TILE_ROWS = 624
TAIL_ROWS = N - (NS - 1) * TILE_ROWS
EPS = 1e-5


def _fill2d(ref, nrows, ncols, val):
    vec = jnp.full((16,), val, jnp.float32)

    def row(i, _):
        def col(j, _):
            ref[i, pl.ds(j * 16, 16)] = vec
            return 0

        lax.fori_loop(0, ncols // 16, col, 0)
        return 0

    lax.fori_loop(0, nrows, row, 0)


def _zero_stripe(src_buf, sh, base, s):
    for off in (0, 128, 256, 384):
        pltpu.sync_copy(src_buf, sh.at[pl.ds(base + off, 128)])
    pltpu.sync_copy(src_buf.at[pl.ds(0, 112)], sh.at[pl.ds(base + 512, 112)])

    @pl.when(s == NS - 1)
    def _():
        pltpu.sync_copy(src_buf.at[pl.ds(0, 24)],
                        sh.at[pl.ds(NS * TILE_ROWS, 24)])


def _write_stripe(sh, out, base, out_row0, s):
    out_base = pl.multiple_of(out_row0 + base, 8)

    @pl.when(s < NS - 1)
    def _():
        pltpu.sync_copy(sh.at[pl.ds(base, TILE_ROWS)],
                        out.at[pl.ds(out_base, TILE_ROWS)])

    @pl.when(s == NS - 1)
    def _():
        pltpu.sync_copy(sh.at[pl.ds(base, TAIL_ROWS)],
                        out.at[pl.ds(out_base, TAIL_ROWS)])


def _cnt_body(dst2, cnt_out, cnt_sh, didx2, ones, ssem):
    c = lax.axis_index("c")
    s = lax.axis_index("s")
    base = pl.multiple_of(s * TILE_ROWS, 8)

    pltpu.sync_copy(dst2.at[pl.ds(s * CPT, CPT)], didx2)
    _fill2d(ones, CHUNK, DH, 0.0)
    _zero_stripe(ones, cnt_sh, base, s)
    _fill2d(ones, CHUNK, DH, 1.0)
    plsc.subcore_barrier()

    def group(g, _):
        descs = [
            pltpu.async_copy(ones, cnt_sh.at[didx2.at[2 * (20 * g + j) + c]],
                             ssem, add=True)
            for j in range(20)
        ]
        for dsc in descs:
            dsc.wait()
        return 0

    lax.fori_loop(0, CPT // 40, group, 0)
    plsc.subcore_barrier()
    _write_stripe(cnt_sh, cnt_out, base, c * N, s)


_sc_cnt = functools.partial(
    pl.kernel,
    out_type=jax.ShapeDtypeStruct((NC * N, DH), jnp.float32),
    mesh=plsc.VectorSubcoreMesh(core_axis_name="c", subcore_axis_name="s",
                                num_cores=NC, num_subcores=NS),
    scratch_types=[
        pltpu.VMEM_SHARED((N + 8, DH), jnp.float32),
        pltpu.VMEM((CPT, CHUNK), jnp.int32),
        pltpu.VMEM((CHUNK, DH), jnp.float32),
        pltpu.SemaphoreType.DMA,
    ],
)(_cnt_body)


def _agg_body(x0, x1, src2, dst2, agg_out, agg_sh, sidx2, didx2,
              rows0, rows1, sem0, sem1):
    c = lax.axis_index("c")
    s = lax.axis_index("s")
    base = pl.multiple_of(s * TILE_ROWS, 8)

    _fill2d(rows0, CHUNK, DH, 0.0)
    _zero_stripe(rows0, agg_sh, base, s)
    plsc.subcore_barrier()

    def work(xh):
        for h in range(CPT // SEG):
            row0 = pl.multiple_of(s * CPT + h * SEG, 8)
            pltpu.sync_copy(src2.at[pl.ds(row0, SEG)], sidx2)
            pltpu.sync_copy(dst2.at[pl.ds(row0, SEG)], didx2)
            pltpu.async_copy(xh.at[sidx2.at[0]], rows0, sem0)

            def step(k2, _):
                k = 2 * k2
                pltpu.async_copy(xh.at[sidx2.at[k + 1]], rows1, sem1)
                pltpu.make_async_copy(xh.at[sidx2.at[0]], rows0, sem0).wait()
                pltpu.sync_copy(rows0, agg_sh.at[didx2.at[k]], add=True)

                @pl.when(k + 2 < SEG)
                def _():
                    pltpu.async_copy(xh.at[sidx2.at[k + 2]], rows0, sem0)

                pltpu.make_async_copy(xh.at[sidx2.at[0]], rows1, sem1).wait()
                pltpu.sync_copy(rows1, agg_sh.at[didx2.at[k + 1]], add=True)
                return 0

            lax.fori_loop(0, SEG // 2, step, 0)

    @pl.when(c == 0)
    def _():
        work(x0)

    @pl.when(c == 1)
    def _():
        work(x1)

    plsc.subcore_barrier()
    _write_stripe(agg_sh, agg_out, base, c * N, s)


_sc_agg = functools.partial(
    pl.kernel,
    out_type=jax.ShapeDtypeStruct((NC * N, DH), jnp.float32),
    mesh=plsc.VectorSubcoreMesh(core_axis_name="c", subcore_axis_name="s",
                                num_cores=NC, num_subcores=NS),
    scratch_types=[
        pltpu.VMEM_SHARED((N + 8, DH), jnp.float32),
        pltpu.VMEM((SEG, CHUNK), jnp.int32),
        pltpu.VMEM((SEG, CHUNK), jnp.int32),
        pltpu.VMEM((CHUNK, DH), jnp.float32),
        pltpu.VMEM((CHUNK, DH), jnp.float32),
        pltpu.SemaphoreType.DMA,
        pltpu.SemaphoreType.DMA,
    ],
)(_agg_body)


NB = 5
BR = N // NB


def _tcr_body(x_ref, wr_ref, bl_ref, r_ref):
    r_ref[...] = jnp.dot(x_ref[...], wr_ref[...],
                         preferred_element_type=jnp.float32) + bl_ref[...]


def _tc_body(r_ref, a0_ref, a1_ref, cnt0_ref, cnt1_ref, wl_ref, g_ref, b_ref,
             o_ref, h_scr, sum_scr, sq_scr):
    p = pl.program_id(0)
    i = pl.program_id(1)

    @pl.when(p == 0)
    def _():
        cnt = cnt0_ref[:, 0:1] + cnt1_ref[:, 0:1]
        inv = 1.0 / jnp.maximum(cnt, 1.0)
        t = jnp.dot(a0_ref[...], wl_ref[0:DH, :],
                    preferred_element_type=jnp.float32)
        t = t + jnp.dot(a1_ref[...], wl_ref[DH:D, :],
                        preferred_element_type=jnp.float32)
        h = t * inv + r_ref[...]
        h_scr[pl.ds(i * BR, BR), :] = h
        csum = jnp.sum(h, axis=0, keepdims=True)
        csq = jnp.sum(h * h, axis=0, keepdims=True)

        @pl.when(i == 0)
        def _():
            sum_scr[...] = csum
            sq_scr[...] = csq

        @pl.when(i > 0)
        def _():
            sum_scr[...] = sum_scr[...] + csum
            sq_scr[...] = sq_scr[...] + csq

    @pl.when(p == 1)
    def _():
        mu = sum_scr[...] * (1.0 / N)
        var = sq_scr[...] * (1.0 / N) - mu * mu
        scale = lax.rsqrt(var + EPS) * g_ref[...]
        h = h_scr[pl.ds(i * BR, BR), :]
        o_ref[...] = jnp.maximum((h - mu) * scale + b_ref[...], 0.0)


def kernel(x, edge_index, W_l, b_l, W_r, gamma, beta):
    pad = ET - E
    src = jnp.concatenate([edge_index[0], jnp.zeros((pad,), jnp.int32)])
    dst = jnp.concatenate(
        [edge_index[1], N + (jnp.arange(pad, dtype=jnp.int32) % 8)])
    src2 = src.reshape(ET // CHUNK, CHUNK)
    dst2 = dst.reshape(ET // CHUNK, CHUNK)
    x0 = x[:, :DH]
    x1 = x[:, DH:]
    cnt = _sc_cnt(dst2)
    agg = _sc_agg(x0, x1, src2, dst2)
    row_blk = lambda p, i: (i, 0)
    r = pl.pallas_call(
        _tcr_body,
        grid=(NB,),
        in_specs=[
            pl.BlockSpec((BR, D), lambda i: (i, 0)),
            pl.BlockSpec((D, D), lambda i: (0, 0)),
            pl.BlockSpec((1, D), lambda i: (0, 0)),
        ],
        out_specs=pl.BlockSpec((BR, D), lambda i: (i, 0)),
        out_shape=jax.ShapeDtypeStruct((N, D), jnp.float32),
    )(x, W_r, b_l.reshape(1, D))
    return pl.pallas_call(
        _tc_body,
        grid=(2, NB),
        in_specs=[
            pl.BlockSpec((BR, D), row_blk),
            pl.BlockSpec((BR, DH), row_blk),
            pl.BlockSpec((BR, DH), lambda p, i: (NB + i, 0)),
            pl.BlockSpec((BR, DH), row_blk),
            pl.BlockSpec((BR, DH), lambda p, i: (NB + i, 0)),
            pl.BlockSpec((D, D), lambda p, i: (0, 0)),
            pl.BlockSpec((1, D), lambda p, i: (0, 0)),
            pl.BlockSpec((1, D), lambda p, i: (0, 0)),
        ],
        out_specs=pl.BlockSpec((BR, D), row_blk),
        out_shape=jax.ShapeDtypeStruct((N, D), jnp.float32),
        scratch_shapes=[
            pltpu.VMEM((N, D), jnp.float32),
            pltpu.VMEM((1, D), jnp.float32),
            pltpu.VMEM((1, D), jnp.float32),
        ],
    )(r, agg, agg, cnt, cnt, W_l, gamma.reshape(1, D), beta.reshape(1, D))
